# baseline (speedup 1.0000x reference)
import functools

import jax
import jax.numpy as jnp
from jax import lax
from jax.experimental import pallas as pl
from jax.experimental.pallas import tpu as pltpu
from jax.experimental.pallas import tpu_sc as plsc

CONTEXT_LEN = 3
EMBED_DIM = 16
HIDDEN_DIM = 100
VOCAB = 100000
BATCH = 1024
N_IDX = BATCH * CONTEXT_LEN

TV = 4096
GRID_V = (VOCAB + TV - 1) // TV


@functools.cache
def _build_sc_gather():
    info = plsc.get_sparse_core_info()
    nc, ns = info.num_cores, info.num_subcores
    nw = nc * ns
    b_per_w = N_IDX // nw
    mesh = plsc.VectorSubcoreMesh(core_axis_name="c", subcore_axis_name="s")

    @functools.partial(
        pl.kernel,
        mesh=mesh,
        out_type=jax.ShapeDtypeStruct((N_IDX, EMBED_DIM), jnp.float32),
        scratch_types=[
            pltpu.VMEM((b_per_w,), jnp.int32),
            pltpu.VMEM((b_per_w, EMBED_DIM), jnp.float32),
            pltpu.SemaphoreType.DMA,
        ],
        compiler_params=pltpu.CompilerParams(use_tc_tiling_on_sc=False),
    )
    def sc_gather(table_hbm, idx_hbm, out_hbm, idx_v, rows_v, sem):
        wid = lax.axis_index("s") * nc + lax.axis_index("c")
        base = wid * b_per_w
        pltpu.sync_copy(idx_hbm.at[pl.ds(base, b_per_w)], idx_v)
        pltpu.async_copy(table_hbm.at[idx_v], rows_v, sem).wait()
        pltpu.sync_copy(rows_v, out_hbm.at[pl.ds(base, b_per_w)])

    return sc_gather


N_Q = 4
QW = TV // N_Q
GRID_M = VOCAB // TV
TAIL_BLK = GRID_M
LAST = GRID_M - 1


def _mlp_body(e_ref, w1_ref, b1_ref, w2_ref, b2_ref, out_ref,
              buf_a, buf_b, sems):
    j = pl.program_id(0)
    z1 = jnp.dot(e_ref[...], w1_ref[...],
                 preferred_element_type=jnp.float32) + b1_ref[...]
    a1 = jnp.tanh(z1)
    tile = jnp.dot(a1, w2_ref[...],
                   preferred_element_type=jnp.float32) + b2_ref[...]

    def q_copy(buf, slot, q, col):
        return pltpu.make_async_copy(
            buf.at[:, pl.ds(q * QW, QW)],
            out_ref.at[:, pl.ds(col, QW)],
            sems.at[slot, q])

    is_a = j % 2 == 0

    @pl.when(jnp.logical_and(j >= 2, is_a))
    def _():
        for q in range(N_Q):
            q_copy(buf_a, 0, q, (j - 2) * TV + q * QW).wait()

    @pl.when(jnp.logical_and(j >= 2, jnp.logical_not(is_a)))
    def _():
        for q in range(N_Q):
            q_copy(buf_b, 1, q, (j - 2) * TV + q * QW).wait()

    @pl.when(is_a)
    def _():
        buf_a[...] = tile
        for q in range(N_Q):
            q_copy(buf_a, 0, q, j * TV + q * QW).start()

    @pl.when(jnp.logical_not(is_a))
    def _():
        buf_b[...] = tile
        for q in range(N_Q):
            q_copy(buf_b, 1, q, j * TV + q * QW).start()

    @pl.when(j == LAST)
    def _():
        for q in range(N_Q):
            q_copy(buf_a, 0, q, (LAST - 1) * TV + q * QW).wait()
        for q in range(N_Q):
            q_copy(buf_b, 1, q, LAST * TV + q * QW).wait()


def _tail_body(prev_ref, e_ref, w1_ref, b1_ref, w2_ref, b2_ref, out_ref):
    del prev_ref
    z1 = jnp.dot(e_ref[...], w1_ref[...],
                 preferred_element_type=jnp.float32) + b1_ref[...]
    a1 = jnp.tanh(z1)
    out_ref[...] = jnp.dot(a1, w2_ref[...],
                           preferred_element_type=jnp.float32) + b2_ref[...]


def _mlp(e, W1, b1, W2, b2):
    d_in = CONTEXT_LEN * EMBED_DIM
    b1r = b1.reshape(1, HIDDEN_DIM)
    b2r = b2.reshape(1, VOCAB)
    main = pl.pallas_call(
        _mlp_body,
        grid=(GRID_M,),
        in_specs=[
            pl.BlockSpec((BATCH, d_in), lambda j: (0, 0)),
            pl.BlockSpec((d_in, HIDDEN_DIM), lambda j: (0, 0)),
            pl.BlockSpec((1, HIDDEN_DIM), lambda j: (0, 0)),
            pl.BlockSpec((HIDDEN_DIM, TV), lambda j: (0, j)),
            pl.BlockSpec((1, TV), lambda j: (0, j)),
        ],
        out_specs=pl.BlockSpec(memory_space=pl.ANY),
        out_shape=jax.ShapeDtypeStruct((BATCH, VOCAB), jnp.float32),
        scratch_shapes=[
            pltpu.VMEM((BATCH, TV), jnp.float32),
            pltpu.VMEM((BATCH, TV), jnp.float32),
            pltpu.SemaphoreType.DMA((2, N_Q)),
        ],
    )(e, W1, b1r, W2, b2r)
    return pl.pallas_call(
        _tail_body,
        grid=(1,),
        in_specs=[
            pl.BlockSpec(memory_space=pl.ANY),
            pl.BlockSpec((BATCH, d_in), lambda j: (0, 0)),
            pl.BlockSpec((d_in, HIDDEN_DIM), lambda j: (0, 0)),
            pl.BlockSpec((1, HIDDEN_DIM), lambda j: (0, 0)),
            pl.BlockSpec((HIDDEN_DIM, TV), lambda j: (0, TAIL_BLK)),
            pl.BlockSpec((1, TV), lambda j: (0, TAIL_BLK)),
        ],
        out_specs=pl.BlockSpec((BATCH, TV), lambda j: (0, TAIL_BLK)),
        out_shape=jax.ShapeDtypeStruct((BATCH, VOCAB), jnp.float32),
        input_output_aliases={0: 0},
    )(main, e, W1, b1r, W2, b2r)


def kernel(x, embed, W1, b1, W2, b2):
    idx = x.reshape(N_IDX).astype(jnp.int32)
    e = _build_sc_gather()(embed, idx)
    e = e.reshape(BATCH, CONTEXT_LEN * EMBED_DIM)
    return _mlp(e, W1, b1, W2, b2)


# speedup vs baseline: 1.0098x; 1.0098x over previous
import functools

import jax
import jax.numpy as jnp
from jax import lax
from jax.experimental import pallas as pl
from jax.experimental.pallas import tpu as pltpu
from jax.experimental.pallas import tpu_sc as plsc

CONTEXT_LEN = 3
EMBED_DIM = 16
HIDDEN_DIM = 100
VOCAB = 100000
BATCH = 1024
N_IDX = BATCH * CONTEXT_LEN

TV = 4096
GRID_V = (VOCAB + TV - 1) // TV


@functools.cache
def _build_sc_gather():
    info = plsc.get_sparse_core_info()
    nc, ns = info.num_cores, info.num_subcores
    nw = nc * ns
    b_per_w = N_IDX // nw
    mesh = plsc.VectorSubcoreMesh(core_axis_name="c", subcore_axis_name="s")

    @functools.partial(
        pl.kernel,
        mesh=mesh,
        out_type=jax.ShapeDtypeStruct((N_IDX, EMBED_DIM), jnp.float32),
        scratch_types=[
            pltpu.VMEM((b_per_w,), jnp.int32),
            pltpu.VMEM((b_per_w, EMBED_DIM), jnp.float32),
            pltpu.SemaphoreType.DMA,
        ],
        compiler_params=pltpu.CompilerParams(use_tc_tiling_on_sc=False),
    )
    def sc_gather(table_hbm, idx_hbm, out_hbm, idx_v, rows_v, sem):
        wid = lax.axis_index("s") * nc + lax.axis_index("c")
        base = wid * b_per_w
        pltpu.sync_copy(idx_hbm.at[pl.ds(base, b_per_w)], idx_v)
        pltpu.async_copy(table_hbm.at[idx_v], rows_v, sem).wait()
        pltpu.sync_copy(rows_v, out_hbm.at[pl.ds(base, b_per_w)])

    return sc_gather


def _mlp_body(e_ref, w1_ref, b1_ref, w2_ref, b2_ref, out_ref, a1_ref):
    @pl.when(pl.program_id(0) == 0)
    def _():
        z1 = jnp.dot(e_ref[...], w1_ref[...],
                     preferred_element_type=jnp.float32) + b1_ref[...]
        a1_ref[...] = jnp.tanh(z1)

    out_ref[...] = jnp.dot(a1_ref[...], w2_ref[...],
                           preferred_element_type=jnp.float32) + b2_ref[...]


def _mlp(e, W1, b1, W2, b2):
    d_in = CONTEXT_LEN * EMBED_DIM
    return pl.pallas_call(
        _mlp_body,
        grid=(GRID_V,),
        in_specs=[
            pl.BlockSpec((BATCH, d_in), lambda j: (0, 0)),
            pl.BlockSpec((d_in, HIDDEN_DIM), lambda j: (0, 0)),
            pl.BlockSpec((1, HIDDEN_DIM), lambda j: (0, 0)),
            pl.BlockSpec((HIDDEN_DIM, TV), lambda j: (0, j)),
            pl.BlockSpec((1, TV), lambda j: (0, j)),
        ],
        out_specs=pl.BlockSpec((BATCH, TV), lambda j: (0, j)),
        out_shape=jax.ShapeDtypeStruct((BATCH, VOCAB), jnp.float32),
        scratch_shapes=[pltpu.VMEM((BATCH, HIDDEN_DIM), jnp.float32)],
    )(e, W1, b1.reshape(1, HIDDEN_DIM), W2, b2.reshape(1, VOCAB))


def kernel(x, embed, W1, b1, W2, b2):
    idx = x.reshape(N_IDX).astype(jnp.int32)
    e = _build_sc_gather()(embed, idx)
    e = e.reshape(BATCH, CONTEXT_LEN * EMBED_DIM)
    return _mlp(e, W1, b1, W2, b2)
